# baseline (speedup 1.0000x reference)
import functools

import jax
import jax.numpy as jnp
from jax import lax
from jax.experimental import pallas as pl
from jax.experimental.pallas import tpu as pltpu
from jax.experimental.pallas import tpu_sc as plsc

NC = 2
NS = 16
NW = NC * NS
CHUNK = 88
NBUF = 4
ROW_BLOCK = 1000


def _mm_bias_body(x_ref, w_ref, b_ref, o_ref):
    o_ref[...] = (
        jnp.dot(x_ref[...], w_ref[...], preferred_element_type=jnp.float32)
        + b_ref[...]
    )


def _mm_bias(x, W, b):
    n, d_in = x.shape
    d_out = W.shape[1]
    grid = n // ROW_BLOCK
    return pl.pallas_call(
        _mm_bias_body,
        grid=(grid,),
        in_specs=[
            pl.BlockSpec((ROW_BLOCK, d_in), lambda i: (i, 0)),
            pl.BlockSpec((d_in, d_out), lambda i: (0, 0)),
            pl.BlockSpec((1, d_out), lambda i: (0, 0)),
        ],
        out_specs=pl.BlockSpec((ROW_BLOCK, d_out), lambda i: (i, 0)),
        out_shape=jax.ShapeDtypeStruct((n, d_out), jnp.float32),
    )(x, W, b.reshape(1, d_out))


def _combine_relu_mm_body(acc_ref, w_ref, b_ref, o_ref):
    x1 = jnp.maximum(acc_ref[0] + acc_ref[1], 0.0)
    o_ref[...] = (
        jnp.dot(x1, w_ref[...], preferred_element_type=jnp.float32) + b_ref[...]
    )


def _combine_relu_mm(parts, W, b, n):
    d_in = parts.shape[2]
    d_out = W.shape[1]
    grid = n // ROW_BLOCK
    return pl.pallas_call(
        _combine_relu_mm_body,
        grid=(grid,),
        in_specs=[
            pl.BlockSpec((2, ROW_BLOCK, d_in), lambda i: (0, i, 0)),
            pl.BlockSpec((d_in, d_out), lambda i: (0, 0)),
            pl.BlockSpec((1, d_out), lambda i: (0, 0)),
        ],
        out_specs=pl.BlockSpec((ROW_BLOCK, d_out), lambda i: (i, 0)),
        out_shape=jax.ShapeDtypeStruct((n, d_out), jnp.float32),
    )(parts, W, b.reshape(1, d_out))


def _combine_body(acc_ref, o_ref):
    o_ref[...] = acc_ref[0] + acc_ref[1]


def _combine(parts, n):
    d = parts.shape[2]
    grid = n // ROW_BLOCK
    return pl.pallas_call(
        _combine_body,
        grid=(grid,),
        in_specs=[pl.BlockSpec((2, ROW_BLOCK, d), lambda i: (0, i, 0))],
        out_specs=pl.BlockSpec((ROW_BLOCK, d), lambda i: (i, 0)),
        out_shape=jax.ShapeDtypeStruct((n, d), jnp.float32),
    )(parts)


def _acc_rows(n):
    return ((n + 1 + NS * 8 - 1) // (NS * 8)) * (NS * 8)


GROUP = 8


def _make_sc_scatter(n, d, ch_per_tile):
    acc_rows = _acc_rows(n)
    per_tile = acc_rows // NS
    assert ch_per_tile % GROUP == 0
    mesh = plsc.VectorSubcoreMesh(core_axis_name="c", subcore_axis_name="s")

    @functools.partial(
        pl.kernel,
        out_type=jax.ShapeDtypeStruct((NC, acc_rows, d), jnp.float32),
        mesh=mesh,
        scratch_types=[
            pltpu.VMEM((2, GROUP, CHUNK), jnp.int32),
            pltpu.VMEM((2, GROUP, CHUNK), jnp.int32),
            pltpu.VMEM((CHUNK, d), jnp.float32),
            pltpu.VMEM((CHUNK, d), jnp.float32),
            pltpu.VMEM((CHUNK, d), jnp.float32),
            pltpu.VMEM((CHUNK, d), jnp.float32),
            pltpu.VMEM_SHARED((acc_rows, d), jnp.float32),
            pltpu.SemaphoreType.DMA,
            pltpu.SemaphoreType.DMA,
            pltpu.SemaphoreType.DMA,
            pltpu.SemaphoreType.DMA,
            pltpu.SemaphoreType.DMA,
            pltpu.SemaphoreType.DMA,
            pltpu.SemaphoreType.DMA,
            pltpu.SemaphoreType.DMA,
            pltpu.SemaphoreType.DMA,
        ],
    )
    def sc_scatter(support_hbm, src_hbm, dst_hbm, out_hbm,
                   srci, dsti, rb0, rb1, rb2, rb3, acc_sh,
                   g0, g1, g2, g3, s0, s1, s2, s3, isem):
        c = lax.axis_index("c")
        s = lax.axis_index("s")
        t = c * NS + s
        bufs = (rb0, rb1, rb2, rb3)
        gsem = (g0, g1, g2, g3)
        ssem = (s0, s1, s2, s3)
        ngroups = ch_per_tile // GROUP

        def drain(sem, buf):
            pltpu.make_async_copy(
                support_hbm.at[pl.ds(0, CHUNK)], buf, sem).wait()

        def zbody(i, _):
            r = i // (d // 16)
            col = (i % (d // 16)) * 16
            rb0[r, pl.ds(col, 16)] = jnp.zeros((16,), jnp.float32)
            return ()
        lax.fori_loop(0, CHUNK * (d // 16), zbody, ())
        zoffs = [(k * CHUNK, min(CHUNK, per_tile - k * CHUNK))
                 for k in range(-(-per_tile // CHUNK))]
        zd = [
            pltpu.async_copy(
                rb0.at[pl.ds(0, nr)],
                acc_sh.at[pl.ds(s * per_tile + r0, nr)], g0)
            for r0, nr in zoffs
        ]
        pltpu.sync_copy(src_hbm.at[t, pl.ds(0, GROUP)], srci.at[0])
        pltpu.sync_copy(dst_hbm.at[t, pl.ds(0, GROUP)], dsti.at[0])
        for dsc in zd:
            dsc.wait()
        plsc.subcore_barrier()

        for m in range(NBUF - 1):
            pltpu.async_copy(support_hbm.at[srci.at[0, m]], bufs[m], gsem[m])

        def group(gi, _):
            b = gi % 2
            nb = 1 - b

            @pl.when(gi + 1 < ngroups)
            def _prefetch():
                off = (gi + 1) * GROUP
                pltpu.async_copy(
                    src_hbm.at[t, pl.ds(off, GROUP)], srci.at[nb], isem)
                pltpu.async_copy(
                    dst_hbm.at[t, pl.ds(off, GROUP)], dsti.at[nb], isem)

            for jj in range(GROUP):
                r = jj % NBUF
                pr = (jj - 1) % NBUF
                drain(gsem[r], bufs[r])
                pltpu.async_copy(
                    bufs[r], acc_sh.at[dsti.at[b, jj]], ssem[r], add=True)
                if jj == 0:
                    @pl.when(gi > 0)
                    def _d0():
                        drain(ssem[pr], bufs[pr])
                else:
                    drain(ssem[pr], bufs[pr])
                la = jj + NBUF - 1
                if la < GROUP:
                    pltpu.async_copy(
                        support_hbm.at[srci.at[b, la]], bufs[pr], gsem[pr])
                else:
                    if la == GROUP:
                        @pl.when(gi + 1 < ngroups)
                        def _drain_prefetch():
                            pltpu.make_async_copy(
                                src_hbm.at[t, pl.ds(0, GROUP)], srci.at[nb],
                                isem).wait()
                            pltpu.make_async_copy(
                                dst_hbm.at[t, pl.ds(0, GROUP)], dsti.at[nb],
                                isem).wait()

                    @pl.when(gi + 1 < ngroups)
                    def _cross_gather():
                        pltpu.async_copy(
                            support_hbm.at[srci.at[nb, la - GROUP]], bufs[pr],
                            gsem[pr])
            return ()
        lax.fori_loop(0, ngroups, group, ())
        drain(ssem[(GROUP - 1) % NBUF], bufs[(GROUP - 1) % NBUF])
        plsc.subcore_barrier()

        pend = [None, None]
        for k, (r0, nr) in enumerate(zoffs):
            p = k & 1
            base = s * per_tile + r0
            if pend[p] is not None:
                pend[p].wait()
            pltpu.sync_copy(acc_sh.at[pl.ds(base, nr)], bufs[p].at[pl.ds(0, nr)])
            pend[p] = pltpu.async_copy(
                bufs[p].at[pl.ds(0, nr)], out_hbm.at[c].at[pl.ds(base, nr)],
                gsem[p])
        for p in (0, 1):
            if pend[p] is not None:
                pend[p].wait()

    return sc_scatter


def kernel(x, adj, W1, b1, W2, b2):
    n, d = x.shape
    e = adj.shape[1]
    ch_per_tile = -(-e // (NW * CHUNK))
    ch_per_tile = ((ch_per_tile + GROUP - 1) // GROUP) * GROUP
    e_pad = NW * ch_per_tile * CHUNK

    src = adj[0].astype(jnp.int32)
    dst = adj[1].astype(jnp.int32)
    pad = e_pad - e
    if pad:
        src = jnp.concatenate([src, jnp.zeros((pad,), jnp.int32)])
        dst = jnp.concatenate([dst, jnp.full((pad,), n, jnp.int32)])
    src_t = src.reshape(NW, ch_per_tile, CHUNK)
    dst_t = dst.reshape(NW, ch_per_tile, CHUNK)

    sc_scatter = _make_sc_scatter(n, d, ch_per_tile)

    support1 = _mm_bias(x, W1, b1)
    parts1 = sc_scatter(support1, src_t, dst_t)
    support2 = _combine_relu_mm(parts1, W2, b2, n)
    parts2 = sc_scatter(support2, src_t, dst_t)
    return _combine(parts2, n)


# speedup vs baseline: 2.6050x; 2.6050x over previous
import functools

import jax
import jax.numpy as jnp
from jax import lax
from jax.experimental import pallas as pl
from jax.experimental.pallas import tpu as pltpu
from jax.experimental.pallas import tpu_sc as plsc

NC = 2
NS = 16
NW = NC * NS
CHUNK = 64
NBUF = 3
ROW_BLOCK = 1000


def _mm_bias_body(x_ref, w_ref, b_ref, o_ref):
    o_ref[...] = (
        jnp.dot(x_ref[...], w_ref[...], preferred_element_type=jnp.float32)
        + b_ref[...]
    )


def _mm_bias(x, W, b):
    n, d_in = x.shape
    d_out = W.shape[1]
    grid = n // ROW_BLOCK
    return pl.pallas_call(
        _mm_bias_body,
        grid=(grid,),
        in_specs=[
            pl.BlockSpec((ROW_BLOCK, d_in), lambda i: (i, 0)),
            pl.BlockSpec((d_in, d_out), lambda i: (0, 0)),
            pl.BlockSpec((1, d_out), lambda i: (0, 0)),
        ],
        out_specs=pl.BlockSpec((ROW_BLOCK, d_out), lambda i: (i, 0)),
        out_shape=jax.ShapeDtypeStruct((n, d_out), jnp.float32),
    )(x, W, b.reshape(1, d_out))


def _combine_relu_mm_body(acc_ref, w_ref, b_ref, o_ref):
    x1 = jnp.maximum(acc_ref[0] + acc_ref[1], 0.0)
    o_ref[...] = (
        jnp.dot(x1, w_ref[...], preferred_element_type=jnp.float32) + b_ref[...]
    )


def _combine_relu_mm(parts, W, b, n):
    d_in = parts.shape[2]
    d_out = W.shape[1]
    grid = n // ROW_BLOCK
    return pl.pallas_call(
        _combine_relu_mm_body,
        grid=(grid,),
        in_specs=[
            pl.BlockSpec((2, ROW_BLOCK, d_in), lambda i: (0, i, 0)),
            pl.BlockSpec((d_in, d_out), lambda i: (0, 0)),
            pl.BlockSpec((1, d_out), lambda i: (0, 0)),
        ],
        out_specs=pl.BlockSpec((ROW_BLOCK, d_out), lambda i: (i, 0)),
        out_shape=jax.ShapeDtypeStruct((n, d_out), jnp.float32),
    )(parts, W, b.reshape(1, d_out))


def _combine_body(acc_ref, o_ref):
    o_ref[...] = acc_ref[0] + acc_ref[1]


def _combine(parts, n):
    d = parts.shape[2]
    grid = n // ROW_BLOCK
    return pl.pallas_call(
        _combine_body,
        grid=(grid,),
        in_specs=[pl.BlockSpec((2, ROW_BLOCK, d), lambda i: (0, i, 0))],
        out_specs=pl.BlockSpec((ROW_BLOCK, d), lambda i: (i, 0)),
        out_shape=jax.ShapeDtypeStruct((n, d), jnp.float32),
    )(parts)


def _acc_rows(n):
    return ((n + 1 + NS * 8 - 1) // (NS * 8)) * (NS * 8)


def _make_sc_scatter(n, d, ch_per_tile):
    acc_rows = _acc_rows(n)
    per_tile = acc_rows // NS
    assert ch_per_tile % NBUF == 0
    nsteps = ch_per_tile // NBUF
    nch = ch_per_tile
    mesh = plsc.VectorSubcoreMesh(core_axis_name="c", subcore_axis_name="s")

    @functools.partial(
        pl.kernel,
        out_type=jax.ShapeDtypeStruct((NC, acc_rows, d), jnp.float32),
        mesh=mesh,
        scratch_types=[
            pltpu.VMEM((ch_per_tile, 2 * CHUNK), jnp.int32),
            pltpu.VMEM((CHUNK, d), jnp.float32),
            pltpu.VMEM((CHUNK, d), jnp.float32),
            pltpu.VMEM((CHUNK, d), jnp.float32),
            pltpu.VMEM_SHARED((acc_rows, d), jnp.float32),
            pltpu.SemaphoreType.DMA,
            pltpu.SemaphoreType.DMA,
            pltpu.SemaphoreType.DMA,
            pltpu.SemaphoreType.DMA,
            pltpu.SemaphoreType.DMA,
            pltpu.SemaphoreType.DMA,
        ],
    )
    def sc_scatter(support_hbm, idx_hbm, out_hbm,
                   idxp, rb0, rb1, rb2, acc_sh, g0, g1, g2, s0, s1, s2):
        c = lax.axis_index("c")
        s = lax.axis_index("s")
        t = c * NS + s
        bufs = (rb0, rb1, rb2)
        gsem = (g0, g1, g2)
        ssem = (s0, s1, s2)

        def drain(sem, buf):
            pltpu.make_async_copy(
                support_hbm.at[pl.ds(0, CHUNK)], buf, sem).wait()

        def zbody(i, _):
            r = i // (d // 16)
            col = (i % (d // 16)) * 16
            rb0[r, pl.ds(col, 16)] = jnp.zeros((16,), jnp.float32)
            return ()
        lax.fori_loop(0, CHUNK * (d // 16), zbody, ())
        zoffs = [(k * CHUNK, min(CHUNK, per_tile - k * CHUNK))
                 for k in range(-(-per_tile // CHUNK))]
        zd = [
            pltpu.async_copy(
                rb0.at[pl.ds(0, nr)],
                acc_sh.at[pl.ds(s * per_tile + r0, nr)], g0)
            for r0, nr in zoffs
        ]
        pltpu.sync_copy(idx_hbm.at[t], idxp)
        for dsc in zd:
            dsc.wait()
        plsc.subcore_barrier()

        for m in range(NBUF - 1):
            pltpu.async_copy(
                support_hbm.at[idxp.at[m, pl.ds(0, CHUNK)]], bufs[m], gsem[m])

        def step(mi, _):
            base = mi * NBUF
            for k in range(NBUF):
                r = k
                pr = (k - 1) % NBUF
                m = base + k
                drain(gsem[r], bufs[r])
                pltpu.async_copy(
                    bufs[r], acc_sh.at[idxp.at[m, pl.ds(CHUNK, CHUNK)]],
                    ssem[r], add=True)
                if k == 0:
                    @pl.when(mi > 0)
                    def _d0():
                        drain(ssem[pr], bufs[pr])
                else:
                    drain(ssem[pr], bufs[pr])
                if k == 0:
                    pltpu.async_copy(
                        support_hbm.at[idxp.at[m + NBUF - 1, pl.ds(0, CHUNK)]],
                        bufs[pr], gsem[pr])
                else:
                    @pl.when(mi + 1 < nsteps)
                    def _la():
                        pltpu.async_copy(
                            support_hbm.at[
                                idxp.at[m + NBUF - 1, pl.ds(0, CHUNK)]],
                            bufs[pr], gsem[pr])
            return ()
        lax.fori_loop(0, nsteps, step, ())
        drain(ssem[(nch - 1) % NBUF], bufs[(nch - 1) % NBUF])
        plsc.subcore_barrier()

        pend = [None, None]
        for k, (r0, nr) in enumerate(zoffs):
            p = k & 1
            base = s * per_tile + r0
            if pend[p] is not None:
                pend[p].wait()
            pltpu.sync_copy(acc_sh.at[pl.ds(base, nr)], bufs[p].at[pl.ds(0, nr)])
            pend[p] = pltpu.async_copy(
                bufs[p].at[pl.ds(0, nr)], out_hbm.at[c].at[pl.ds(base, nr)],
                gsem[p])
        for p in (0, 1):
            if pend[p] is not None:
                pend[p].wait()

    return sc_scatter


def kernel(x, adj, W1, b1, W2, b2):
    n, d = x.shape
    e = adj.shape[1]
    ch_per_tile = -(-e // (NW * CHUNK))
    ch_per_tile = ((ch_per_tile + NBUF - 1) // NBUF) * NBUF
    e_pad = NW * ch_per_tile * CHUNK

    src = adj[0].astype(jnp.int32)
    dst = adj[1].astype(jnp.int32)
    pad = e_pad - e
    if pad:
        src = jnp.concatenate([src, jnp.zeros((pad,), jnp.int32)])
        dst = jnp.concatenate([dst, jnp.full((pad,), n, jnp.int32)])
    idx_t = jnp.concatenate(
        [src.reshape(NW, ch_per_tile, CHUNK),
         dst.reshape(NW, ch_per_tile, CHUNK)], axis=2)

    sc_scatter = _make_sc_scatter(n, d, ch_per_tile)

    support1 = _mm_bias(x, W1, b1)
    parts1 = sc_scatter(support1, idx_t)
    support2 = _combine_relu_mm(parts1, W2, b2, n)
    parts2 = sc_scatter(support2, idx_t)
    return _combine(parts2, n)
